# baseline (speedup 1.0000x reference)
import jax
import jax.numpy as jnp
from jax.experimental import pallas as pl
from jax.experimental.pallas import tpu as pltpu

B, T, C = 1, 2048, 1024
E, K = 8, 2
FF = 4 * C
CAP = (B * T) // E
FFT = 1024
NFF = FF // FFT
GPAD = 128


def _gate_body(x_ref, wg_ref, bg_ref, o_ref):
    o_ref[...] = jax.nn.sigmoid(
        jnp.dot(x_ref[...], wg_ref[...], preferred_element_type=jnp.float32)
        + bg_ref[...]
    )


def _moe_body(tok_ref, sco_ref, xf_ref, w1_ref, b1_ref, w2_ref, b2_ref,
              o_ref, xs_ref, acc_ref):
    e = pl.program_id(0)
    ff = pl.program_id(1)

    @pl.when(jnp.logical_and(e == 0, ff == 0))
    def _():
        o_ref[...] = jnp.zeros_like(o_ref)

    @pl.when(ff == 0)
    def _():
        acc_ref[...] = jnp.zeros_like(acc_ref)

        def gather(i, carry):
            t = tok_ref[e, i]
            xs_ref[pl.ds(i, 1), :] = xf_ref[pl.ds(t, 1), :]
            return carry

        jax.lax.fori_loop(0, CAP, gather, 0)

    h = jnp.dot(xs_ref[...], w1_ref[0], preferred_element_type=jnp.float32)
    h = jax.nn.gelu(h + b1_ref[0])
    acc_ref[...] += jnp.dot(h, w2_ref[0], preferred_element_type=jnp.float32)

    @pl.when(ff == NFF - 1)
    def _():
        acc_ref[...] += b2_ref[0]

        def scatter(i, carry):
            t = tok_ref[e, i]
            s = sco_ref[e, i]
            o_ref[pl.ds(t, 1), :] += acc_ref[pl.ds(i, 1), :] * s
            return carry

        jax.lax.fori_loop(0, CAP, scatter, 0)


def _routing(scores):
    g_i, idx = jax.lax.top_k(scores, K)
    g_scores = g_i / jnp.sum(g_i, axis=-1, keepdims=True)
    sti = jnp.argsort(-g_scores[:, 0])
    sind = jnp.take_along_axis(idx, sti[:, None], axis=0)
    ssc = jnp.take_along_axis(g_scores, sti[:, None], axis=0)
    flat_ind = jnp.swapaxes(sind, 0, 1).reshape(-1)
    flat_sc = jnp.swapaxes(ssc, 0, 1).reshape(-1)
    oh = jax.nn.one_hot(flat_ind, E, dtype=jnp.int32)
    pie = jnp.cumsum(oh, axis=0) * oh
    tokens_per_expert = jnp.max(pie, axis=0) / (B * T)
    esc = flat_sc[:, None] * oh
    pie_t = jnp.swapaxes(pie.reshape(K, T, E), 0, 1)
    esc_t = jnp.swapaxes(esc.reshape(K, T, E), 0, 1)
    pos_s = jnp.max(pie_t, axis=1) - 1
    sc_s = jnp.max(esc_t, axis=1)
    kept = (pos_s >= 0) & (pos_s < CAP)
    col = jnp.where(kept, pos_s, CAP)
    ee = jnp.broadcast_to(jnp.arange(E)[None, :], (T, E))
    tt = jnp.broadcast_to(sti[:, None], (T, E))
    tok = jnp.zeros((E, CAP + 1), jnp.int32).at[ee, col].set(tt)[:, :CAP]
    sco = jnp.zeros((E, CAP + 1), jnp.float32).at[ee, col].set(
        jnp.where(kept, sc_s, 0.0))[:, :CAP]
    sn = scores / jnp.sum(scores, axis=-1, keepdims=True)
    sn = jnp.take_along_axis(sn, idx, axis=-1)
    ohf = jax.nn.one_hot(idx.reshape(-1), E, dtype=jnp.float32)
    f = jnp.sum(ohf, axis=0) / (B * T)
    p = jnp.sum(ohf * sn.reshape(-1)[:, None], axis=0) / (B * T)
    return tok, sco, tokens_per_expert, f, p


def kernel(x, W_shared, b_shared, W_gate, b_gate, W1, b1, W2, b2):
    xf = x.reshape(T, C)
    Wg = jnp.zeros((C, GPAD), x.dtype).at[:, :E].set(W_gate)
    bg = jnp.zeros((1, GPAD), x.dtype).at[0, :E].set(b_gate)
    scores_pad = pl.pallas_call(
        _gate_body,
        out_shape=jax.ShapeDtypeStruct((T, GPAD), jnp.float32),
    )(xf, Wg, bg)
    scores = scores_pad[:, :E]

    tok, sco, tokens_per_expert, f, p = _routing(scores)

    out = pl.pallas_call(
        _moe_body,
        grid=(E, NFF),
        in_specs=[
            pl.BlockSpec(memory_space=pltpu.SMEM),
            pl.BlockSpec(memory_space=pltpu.SMEM),
            pl.BlockSpec((T, C), lambda e, f_: (0, 0)),
            pl.BlockSpec((1, C, FFT), lambda e, f_: (e, 0, f_)),
            pl.BlockSpec((1, 1, FFT), lambda e, f_: (e, 0, f_)),
            pl.BlockSpec((1, FFT, C), lambda e, f_: (e, f_, 0)),
            pl.BlockSpec((1, 1, C), lambda e, f_: (e, 0, 0)),
        ],
        out_specs=pl.BlockSpec((T, C), lambda e, f_: (0, 0)),
        out_shape=jax.ShapeDtypeStruct((T, C), jnp.float32),
        scratch_shapes=[
            pltpu.VMEM((CAP, C), jnp.float32),
            pltpu.VMEM((CAP, C), jnp.float32),
        ],
        compiler_params=pltpu.CompilerParams(
            dimension_semantics=("arbitrary", "arbitrary")),
    )(tok, sco, xf, W1, b1.reshape(E, 1, FF), W2, b2.reshape(E, 1, C))

    return out.reshape(B, T, C), tokens_per_expert, f, p
